# baseline (speedup 1.0000x reference)
import functools
import math

import jax
import jax.numpy as jnp
from jax import lax
from jax.experimental import pallas as pl
from jax.experimental.pallas import tpu as pltpu
from jax.experimental.pallas import tpu_sc as plsc

D = 128
N_USER = 5000
N_ITEM = 5000
N_EDGES = 320000
NPAD = 5120
HALF = 160
CHUNK = 160
E_PER_W = N_EDGES // 32
E_PER_T = N_EDGES // 16
INV_SQRT_D = 1.0 / math.sqrt(128.0)

_mesh = plsc.VectorSubcoreMesh(core_axis_name="c", subcore_axis_name="s")
_sc_params = pltpu.CompilerParams(needs_layout_passes=False)


@functools.partial(
    pl.kernel,
    mesh=_mesh,
    out_type=[
        jax.ShapeDtypeStruct((NPAD, D), jnp.float32),
        jax.ShapeDtypeStruct((NPAD, D), jnp.float32),
        jax.ShapeDtypeStruct((N_EDGES,), jnp.float32),
    ],
    scratch_types=[
        pltpu.VMEM_SHARED((NPAD, D), jnp.float32),
        pltpu.VMEM_SHARED((NPAD, D), jnp.float32),
        pltpu.VMEM((CHUNK, D), jnp.float32),
        pltpu.VMEM((CHUNK, D), jnp.float32),
        pltpu.VMEM((HALF,), jnp.int32),
        pltpu.VMEM((CHUNK,), jnp.int32),
        pltpu.VMEM((CHUNK,), jnp.int32),
        pltpu.VMEM((CHUNK,), jnp.float32),
        pltpu.SemaphoreType.DMA,
    ],
    compiler_params=_sc_params,
)
def _k1(feat, uids, iids, eu, ev, hs_out, hd_out, s_out,
        hs_sh, hd_sh, rows_a, rows_b, sidx, ub, vb, sb, sem):
    cid = lax.axis_index("c")
    sid = lax.axis_index("s")
    wid = sid * 2 + cid

    for half in range(2):
        base = sid * (2 * HALF) + half * HALF
        pltpu.sync_copy(uids.at[pl.ds(base, HALF)], sidx)
        pltpu.async_copy(feat.at[sidx], rows_a.at[pl.ds(0, HALF)], sem).wait()
        pltpu.sync_copy(rows_a.at[pl.ds(0, HALF)],
                        hs_sh.at[pl.ds(base, HALF)])

        pltpu.sync_copy(iids.at[pl.ds(base, HALF)], sidx)
        pltpu.async_copy(feat.at[sidx], rows_b.at[pl.ds(0, HALF)], sem).wait()
        pltpu.sync_copy(rows_b.at[pl.ds(0, HALF)],
                        hd_sh.at[pl.ds(base, HALF)])

        @pl.when(cid == 0)
        def _():
            pltpu.sync_copy(rows_a.at[pl.ds(0, HALF)],
                            hs_out.at[pl.ds(base, HALF)])
            pltpu.sync_copy(rows_b.at[pl.ds(0, HALF)],
                            hd_out.at[pl.ds(base, HALF)])

    plsc.subcore_barrier()

    lane = lax.iota(jnp.int32, 16)
    n_chunks_total = N_EDGES // CHUNK

    def chunk_body(k, _):
        eb = (k * 32 + wid) * CHUNK
        pltpu.sync_copy(eu.at[pl.ds(eb, CHUNK)], ub)
        pltpu.sync_copy(ev.at[pl.ds(eb, CHUNK)], vb)
        pltpu.async_copy(hs_sh.at[ub], rows_a, sem).wait()
        pltpu.async_copy(hd_sh.at[vb], rows_b, sem).wait()

        def grp_body(g, _):
            e0 = g * 16
            svec = jnp.zeros((16,), jnp.float32)
            for r in range(16):
                acc = rows_a[e0 + r, pl.ds(0, 16)] * rows_b[e0 + r, pl.ds(0, 16)]
                for j in range(1, 8):
                    acc = acc + (rows_a[e0 + r, pl.ds(16 * j, 16)] *
                                 rows_b[e0 + r, pl.ds(16 * j, 16)])
                svec = jnp.where(lane == r, jnp.sum(acc), svec)
            sb[pl.ds(e0, 16)] = svec
            return 0

        lax.fori_loop(0, CHUNK // 16, grp_body, 0)
        pltpu.sync_copy(sb, s_out.at[pl.ds(eb, CHUNK)])
        return 0

    my_chunks = (n_chunks_total - wid + 31) // 32
    lax.fori_loop(0, my_chunks, chunk_body, 0)


def _k2_body(hs, hd, ws, bs, wd, bd, s, fs_out, fd_out, w_out):
    dn = (((1,), (1,)), ((), ()))
    fs = lax.dot_general(hs[...], ws[...], dn,
                         preferred_element_type=jnp.float32,
                         precision=lax.Precision.HIGHEST)
    fs_out[...] = jnp.maximum(fs + bs[...], 0.0)
    fd = lax.dot_general(hd[...], wd[...], dn,
                         preferred_element_type=jnp.float32,
                         precision=lax.Precision.HIGHEST)
    fd_out[...] = jnp.maximum(fd + bd[...], 0.0)

    sv = s[...] * INV_SQRT_D
    m = jnp.max(sv)
    e = jnp.exp(sv - m)
    w_out[...] = e / jnp.sum(e)


_k2 = pl.pallas_call(
    _k2_body,
    out_shape=[
        jax.ShapeDtypeStruct((NPAD, D), jnp.float32),
        jax.ShapeDtypeStruct((NPAD, D), jnp.float32),
        jax.ShapeDtypeStruct((N_EDGES // D, D), jnp.float32),
    ],
)


@functools.partial(
    pl.kernel,
    mesh=_mesh,
    out_type=[
        jax.ShapeDtypeStruct((NPAD, D), jnp.float32),
        jax.ShapeDtypeStruct((NPAD, D), jnp.float32),
    ],
    scratch_types=[
        pltpu.VMEM_SHARED((NPAD, D), jnp.float32),
        pltpu.VMEM_SHARED((NPAD, D), jnp.float32),
        pltpu.VMEM((CHUNK, D), jnp.float32),
        pltpu.VMEM((CHUNK,), jnp.int32),
        pltpu.VMEM((CHUNK,), jnp.int32),
        pltpu.VMEM((CHUNK,), jnp.float32),
        pltpu.SemaphoreType.DMA,
    ],
    compiler_params=_sc_params,
)
def _k3(eu, ev, w, fs, fd, user_out, item_out,
        tab_sh, acc_sh, rows, ub, vb, wb, sem):
    cid = lax.axis_index("c")
    sid = lax.axis_index("s")

    zv = jnp.zeros((16,), jnp.float32)

    def zrow(i, _):
        for j in range(8):
            rows[i, pl.ds(16 * j, 16)] = zv
        return 0

    lax.fori_loop(0, HALF, zrow, 0)
    for half in range(2):
        base = sid * (2 * HALF) + half * HALF
        pltpu.sync_copy(rows.at[pl.ds(0, HALF)],
                        acc_sh.at[pl.ds(base, HALF)])

    for half in range(2):
        base = sid * (2 * HALF) + half * HALF

        @pl.when(cid == 0)
        def _():
            pltpu.sync_copy(fs.at[pl.ds(base, HALF)],
                            rows.at[pl.ds(0, HALF)])

        @pl.when(cid == 1)
        def _():
            pltpu.sync_copy(fd.at[pl.ds(base, HALF)],
                            rows.at[pl.ds(0, HALF)])

        pltpu.sync_copy(rows.at[pl.ds(0, HALF)],
                        tab_sh.at[pl.ds(base, HALF)])

    plsc.subcore_barrier()

    ebase = sid * E_PER_T

    def do_chunks(gidx, scidx):
        def chunk_body(ci, _):
            eb = ebase + ci * CHUNK
            pltpu.sync_copy(eu.at[pl.ds(eb, CHUNK)], ub)
            pltpu.sync_copy(ev.at[pl.ds(eb, CHUNK)], vb)
            pltpu.sync_copy(w.at[pl.ds(eb, CHUNK)], wb)
            pltpu.async_copy(tab_sh.at[gidx], rows, sem).wait()

            def scale_grp(g, _):
                e0 = g * 16
                wv = wb[pl.ds(e0, 16)]
                for l in range(16):
                    we = wv[l]
                    for j in range(8):
                        rows[e0 + l, pl.ds(16 * j, 16)] = (
                            rows[e0 + l, pl.ds(16 * j, 16)] * we)
                return 0

            lax.fori_loop(0, CHUNK // 16, scale_grp, 0)
            pltpu.sync_copy(rows, acc_sh.at[scidx], add=True)
            return 0

        lax.fori_loop(0, E_PER_T // CHUNK, chunk_body, 0)

    @pl.when(cid == 0)
    def _():
        do_chunks(ub, vb)

    @pl.when(cid == 1)
    def _():
        do_chunks(vb, ub)

    plsc.subcore_barrier()

    for half in range(2):
        base = sid * (2 * HALF) + half * HALF
        pltpu.sync_copy(acc_sh.at[pl.ds(base, HALF)],
                        rows.at[pl.ds(0, HALF)])

        @pl.when(cid == 0)
        def _():
            pltpu.sync_copy(rows.at[pl.ds(0, HALF)],
                            item_out.at[pl.ds(base, HALF)])

        @pl.when(cid == 1)
        def _():
            pltpu.sync_copy(rows.at[pl.ds(0, HALF)],
                            user_out.at[pl.ds(base, HALF)])


def kernel(feat, user_ids, item_ids, edge_index, W_src, b_src, W_dst, b_dst):
    uids = jnp.pad(user_ids, (0, NPAD - N_USER))
    iids = jnp.pad(item_ids, (0, NPAD - N_ITEM))
    eu = edge_index[0]
    ev = edge_index[1]

    h_src, h_dst, s = _k1(feat, uids, iids, eu, ev)

    feat_src, feat_dst, w2d = _k2(
        h_src, h_dst, W_src, b_src.reshape(1, D), W_dst, b_dst.reshape(1, D),
        s.reshape(N_EDGES // D, D))
    w = w2d.reshape(N_EDGES)

    e_user, e_item = _k3(eu, ev, w, feat_src, feat_dst)
    return jnp.concatenate([e_user[:N_USER], e_item[:N_ITEM]], axis=0)


# speedup vs baseline: 1.4109x; 1.4109x over previous
import functools
import math

import jax
import jax.numpy as jnp
from jax import lax
from jax.experimental import pallas as pl
from jax.experimental.pallas import tpu as pltpu
from jax.experimental.pallas import tpu_sc as plsc

D = 128
N_USER = 5000
N_ITEM = 5000
N_EDGES = 320000
NPAD = 5120
C1 = 80
C3 = 160
EPAD = N_EDGES + 2 * C3
N1 = N_EDGES // 32 // C1
N3 = N_EDGES // 16 // C3
INV_SQRT_D = 1.0 / math.sqrt(128.0)

_mesh = plsc.VectorSubcoreMesh(core_axis_name="c", subcore_axis_name="s")
_sc_params = pltpu.CompilerParams(needs_layout_passes=False)


@functools.partial(
    pl.kernel,
    mesh=_mesh,
    out_type=[
        jax.ShapeDtypeStruct((NPAD, D), jnp.float32),
        jax.ShapeDtypeStruct((NPAD, D), jnp.float32),
        jax.ShapeDtypeStruct((N_EDGES,), jnp.float32),
    ],
    scratch_types=[
        pltpu.VMEM_SHARED((NPAD, D), jnp.float32),
        pltpu.VMEM_SHARED((NPAD, D), jnp.float32),
        pltpu.VMEM((2 * C1, D), jnp.float32),
        pltpu.VMEM((2 * C1, D), jnp.float32),
        pltpu.VMEM((2 * C1,), jnp.float32),
        pltpu.VMEM((C1,), jnp.int32),
        pltpu.VMEM((C1,), jnp.int32),
        pltpu.VMEM((C1,), jnp.int32),
        pltpu.VMEM((C1,), jnp.int32),
        pltpu.VMEM((C1,), jnp.int32),
        pltpu.SemaphoreType.DMA,
        pltpu.SemaphoreType.DMA,
        pltpu.SemaphoreType.DMA,
        pltpu.SemaphoreType.DMA,
        pltpu.SemaphoreType.DMA,
        pltpu.SemaphoreType.DMA,
        pltpu.SemaphoreType.DMA,
    ],
    compiler_params=_sc_params,
)
def _k1(feat, uids, iids, eu, ev, hs_out, hd_out, s_out,
        hs_sh, hd_sh, ra, rb, sb, ub0, ub1, vb0, vb1,
        sidx, is0, is1, gs0, gs1, ws0, ws1, ssem):
    cid = lax.axis_index("c")
    sid = lax.axis_index("s")
    wid = sid * 2 + cid

    ub = (ub0, ub1)
    vb = (vb0, vb1)
    isem = (is0, is1)
    gsem = (gs0, gs1)
    wsem = (ws0, ws1)

    for q in range(4):
        base = sid * (4 * C1) + q * C1
        pltpu.sync_copy(uids.at[pl.ds(base, C1)], sidx)
        pltpu.async_copy(feat.at[sidx], ra.at[pl.ds(0, C1)], ssem).wait()
        pltpu.sync_copy(ra.at[pl.ds(0, C1)], hs_sh.at[pl.ds(base, C1)])

        pltpu.sync_copy(iids.at[pl.ds(base, C1)], sidx)
        pltpu.async_copy(feat.at[sidx], rb.at[pl.ds(0, C1)], ssem).wait()
        pltpu.sync_copy(rb.at[pl.ds(0, C1)], hd_sh.at[pl.ds(base, C1)])

        @pl.when(cid == 0)
        def _():
            pltpu.sync_copy(ra.at[pl.ds(0, C1)], hs_out.at[pl.ds(base, C1)])
            pltpu.sync_copy(rb.at[pl.ds(0, C1)], hd_out.at[pl.ds(base, C1)])

    plsc.subcore_barrier()

    ebase = wid * (N1 * C1)
    lane = lax.iota(jnp.int32, 16)

    def issue_idx(c, b):
        eb = ebase + c * C1
        pltpu.async_copy(eu.at[pl.ds(eb, C1)], ub[b], isem[b])
        pltpu.async_copy(ev.at[pl.ds(eb, C1)], vb[b], isem[b])

    def wait_idx(b):
        pltpu.make_async_copy(eu.at[pl.ds(0, C1)], ub[b], isem[b]).wait()
        pltpu.make_async_copy(eu.at[pl.ds(0, C1)], vb[b], isem[b]).wait()

    def issue_gather(b, roff):
        pltpu.async_copy(hs_sh.at[ub[b]], ra.at[pl.ds(roff, C1)], gsem[b])
        pltpu.async_copy(hd_sh.at[vb[b]], rb.at[pl.ds(roff, C1)], gsem[b])

    def wait_gather(b):
        pltpu.make_async_copy(feat.at[pl.ds(0, C1)],
                              ra.at[pl.ds(0, C1)], gsem[b]).wait()
        pltpu.make_async_copy(feat.at[pl.ds(0, C1)],
                              rb.at[pl.ds(0, C1)], gsem[b]).wait()

    def issue_wb(c, b, roff):
        eb = ebase + c * C1
        pltpu.async_copy(sb.at[pl.ds(roff, C1)], s_out.at[pl.ds(eb, C1)],
                         wsem[b])

    def wait_wb(b):
        pltpu.make_async_copy(sb.at[pl.ds(0, C1)], s_out.at[pl.ds(0, C1)],
                              wsem[b]).wait()

    issue_idx(0, 0)
    wait_idx(0)
    issue_gather(0, 0)
    issue_idx(1, 1)

    def body(c, _):
        par = c & 1
        roff = par * C1

        @pl.when(par == 0)
        def _():
            wait_gather(0)

        @pl.when(par == 1)
        def _():
            wait_gather(1)

        @pl.when((c >= 2) & (par == 0))
        def _():
            wait_wb(0)

        @pl.when((c >= 2) & (par == 1))
        def _():
            wait_wb(1)

        def grp_body(g, _):
            e0 = roff + g * 16
            svec = jnp.zeros((16,), jnp.float32)
            for r in range(16):
                acc = ra[e0 + r, pl.ds(0, 16)] * rb[e0 + r, pl.ds(0, 16)]
                for j in range(1, 8):
                    acc = acc + (ra[e0 + r, pl.ds(16 * j, 16)] *
                                 rb[e0 + r, pl.ds(16 * j, 16)])
                svec = jnp.where(lane == r, jnp.sum(acc), svec)
            sb[pl.ds(e0, 16)] = svec
            return 0

        lax.fori_loop(0, C1 // 16, grp_body, 0)

        @pl.when(par == 0)
        def _():
            issue_wb(c, 0, 0)
            issue_idx(c + 2, 0)
            wait_idx(1)
            issue_gather(1, C1)

        @pl.when(par == 1)
        def _():
            issue_wb(c, 1, C1)
            issue_idx(c + 2, 1)
            wait_idx(0)
            issue_gather(0, 0)

        return 0

    lax.fori_loop(0, N1, body, 0)

    wait_gather(N1 & 1)
    wait_idx((N1 + 1) & 1)
    wait_wb(0)
    wait_wb(1)


def _k2_body(hs, hd, ws, bs, wd, bd, s, fs_out, fd_out, w_out):
    dn = (((1,), (1,)), ((), ()))
    fs = lax.dot_general(hs[...], ws[...], dn,
                         preferred_element_type=jnp.float32,
                         precision=lax.Precision.HIGHEST)
    fs_out[...] = jnp.maximum(fs + bs[...], 0.0)
    fd = lax.dot_general(hd[...], wd[...], dn,
                         preferred_element_type=jnp.float32,
                         precision=lax.Precision.HIGHEST)
    fd_out[...] = jnp.maximum(fd + bd[...], 0.0)

    sv = s[...] * INV_SQRT_D
    m = jnp.max(sv)
    e = jnp.exp(sv - m)
    w_out[...] = e / jnp.sum(e)


_k2 = pl.pallas_call(
    _k2_body,
    out_shape=[
        jax.ShapeDtypeStruct((NPAD, D), jnp.float32),
        jax.ShapeDtypeStruct((NPAD, D), jnp.float32),
        jax.ShapeDtypeStruct((N_EDGES // D, D), jnp.float32),
    ],
)


@functools.partial(
    pl.kernel,
    mesh=_mesh,
    out_type=[
        jax.ShapeDtypeStruct((NPAD, D), jnp.float32),
        jax.ShapeDtypeStruct((NPAD, D), jnp.float32),
    ],
    scratch_types=[
        pltpu.VMEM_SHARED((NPAD, D), jnp.float32),
        pltpu.VMEM_SHARED((NPAD, D), jnp.float32),
        pltpu.VMEM((2 * C3, D), jnp.float32),
        pltpu.VMEM((2 * C3,), jnp.float32),
        pltpu.VMEM((C3,), jnp.int32),
        pltpu.VMEM((C3,), jnp.int32),
        pltpu.VMEM((C3,), jnp.int32),
        pltpu.VMEM((C3,), jnp.int32),
        pltpu.VMEM((C3,), jnp.int32),
        pltpu.VMEM((C3,), jnp.int32),
        pltpu.SemaphoreType.DMA,
        pltpu.SemaphoreType.DMA,
        pltpu.SemaphoreType.DMA,
        pltpu.SemaphoreType.DMA,
        pltpu.SemaphoreType.DMA,
        pltpu.SemaphoreType.DMA,
    ],
    compiler_params=_sc_params,
)
def _k3(eu, ev, w, fs, fd, user_out, item_out,
        tab_sh, acc_sh, rows, wgt, gb0, gb1, cb0, cb1, sx0, sx1,
        is0, is1, gs0, gs1, ss0, ss1):
    cid = lax.axis_index("c")
    sid = lax.axis_index("s")

    gb = (gb0, gb1)
    cb = (cb0, cb1)
    sx = (sx0, sx1)
    isem = (is0, is1)
    gsem = (gs0, gs1)
    ssem = (ss0, ss1)

    zv = jnp.zeros((16,), jnp.float32)

    def zrow(i, _):
        for j in range(8):
            rows[i, pl.ds(16 * j, 16)] = zv
        return 0

    lax.fori_loop(0, 2 * C3, zrow, 0)
    abase = sid * (2 * C3)
    pltpu.sync_copy(rows, acc_sh.at[pl.ds(abase, 2 * C3)])

    for half in range(2):
        base = abase + half * C3

        @pl.when(cid == 0)
        def _():
            pltpu.sync_copy(fs.at[pl.ds(base, C3)], rows.at[pl.ds(0, C3)])

        @pl.when(cid == 1)
        def _():
            pltpu.sync_copy(fd.at[pl.ds(base, C3)], rows.at[pl.ds(0, C3)])

        pltpu.sync_copy(rows.at[pl.ds(0, C3)], tab_sh.at[pl.ds(base, C3)])

    plsc.subcore_barrier()

    ebase = sid * (N3 * C3)

    def issue_idx(c, b):
        eb = ebase + c * C3

        @pl.when(cid == 0)
        def _():
            pltpu.async_copy(eu.at[pl.ds(eb, C3)], gb[b], isem[b])
            pltpu.async_copy(ev.at[pl.ds(eb, C3)], cb[b], isem[b])

        @pl.when(cid == 1)
        def _():
            pltpu.async_copy(ev.at[pl.ds(eb, C3)], gb[b], isem[b])
            pltpu.async_copy(eu.at[pl.ds(eb, C3)], cb[b], isem[b])

        pltpu.async_copy(w.at[pl.ds(eb, C3)], wgt.at[pl.ds(b * C3, C3)],
                         isem[b])

    def wait_idx(b):
        pltpu.make_async_copy(eu.at[pl.ds(0, C3)], gb[b], isem[b]).wait()
        pltpu.make_async_copy(eu.at[pl.ds(0, C3)], cb[b], isem[b]).wait()
        pltpu.make_async_copy(w.at[pl.ds(0, C3)], wgt.at[pl.ds(0, C3)],
                              isem[b]).wait()

    def issue_gather(b, roff):
        pltpu.async_copy(tab_sh.at[gb[b]], rows.at[pl.ds(roff, C3)], gsem[b])

    def wait_gather(b):
        pltpu.make_async_copy(fs.at[pl.ds(0, C3)], rows.at[pl.ds(0, C3)],
                              gsem[b]).wait()

    def issue_scat(b, roff):
        pltpu.async_copy(rows.at[pl.ds(roff, C3)], acc_sh.at[sx[b]], ssem[b],
                         add=True)

    def wait_scat(b):
        pltpu.make_async_copy(fs.at[pl.ds(0, C3)], acc_sh.at[pl.ds(0, C3)],
                              ssem[b]).wait()

    issue_idx(0, 0)
    wait_idx(0)
    issue_gather(0, 0)
    issue_idx(1, 1)

    def body(c, _):
        par = c & 1
        roff = par * C3

        @pl.when(par == 0)
        def _():
            wait_gather(0)

        @pl.when(par == 1)
        def _():
            wait_gather(1)

        def sgrp(g, _):
            o = g * 16

            @pl.when(par == 0)
            def _():
                sx0[pl.ds(o, 16)] = cb0[pl.ds(o, 16)]

            @pl.when(par == 1)
            def _():
                sx1[pl.ds(o, 16)] = cb1[pl.ds(o, 16)]

            e0 = roff + o
            wv = wgt[pl.ds(e0, 16)]
            for l in range(16):
                we = wv[l]
                for j in range(8):
                    rows[e0 + l, pl.ds(16 * j, 16)] = (
                        rows[e0 + l, pl.ds(16 * j, 16)] * we)
            return 0

        lax.fori_loop(0, C3 // 16, sgrp, 0)

        @pl.when(par == 0)
        def _():
            issue_scat(0, 0)
            issue_idx(c + 2, 0)
            wait_idx(1)

            @pl.when(c >= 1)
            def _():
                wait_scat(1)

            issue_gather(1, C3)

        @pl.when(par == 1)
        def _():
            issue_scat(1, C3)
            issue_idx(c + 2, 1)
            wait_idx(0)
            wait_scat(0)
            issue_gather(0, 0)

        return 0

    lax.fori_loop(0, N3, body, 0)

    wait_gather(N3 & 1)
    wait_idx((N3 + 1) & 1)
    wait_scat(0)
    wait_scat(1)

    plsc.subcore_barrier()

    pltpu.sync_copy(acc_sh.at[pl.ds(abase, 2 * C3)], rows)

    @pl.when(cid == 0)
    def _():
        pltpu.sync_copy(rows, item_out.at[pl.ds(abase, 2 * C3)])

    @pl.when(cid == 1)
    def _():
        pltpu.sync_copy(rows, user_out.at[pl.ds(abase, 2 * C3)])


def kernel(feat, user_ids, item_ids, edge_index, W_src, b_src, W_dst, b_dst):
    uids = jnp.pad(user_ids, (0, NPAD - N_USER))
    iids = jnp.pad(item_ids, (0, NPAD - N_ITEM))
    eu = jnp.pad(edge_index[0], (0, EPAD - N_EDGES))
    ev = jnp.pad(edge_index[1], (0, EPAD - N_EDGES))

    h_src, h_dst, s = _k1(feat, uids, iids, eu, ev)

    feat_src, feat_dst, w2d = _k2(
        h_src, h_dst, W_src, b_src.reshape(1, D), W_dst, b_dst.reshape(1, D),
        s.reshape(N_EDGES // D, D))
    w = jnp.pad(w2d.reshape(N_EDGES), (0, EPAD - N_EDGES))

    e_user, e_item = _k3(eu, ev, w, feat_src, feat_dst)
    return jnp.concatenate([e_user[:N_USER], e_item[:N_ITEM]], axis=0)
